# Initial kernel scaffold; baseline (speedup 1.0000x reference)
import jax
import jax.numpy as jnp
from jax.experimental import pallas as pl


def kernel(x, edge_index, Wself0, Wneigh0, b0, Wself1, Wneigh1, b1, Wself2, Wneigh2, b2):
    raise NotImplementedError("write your pallas kernel here")


# speedup vs baseline: 4.6091x; 4.6091x over previous
import functools

import jax
import jax.numpy as jnp
from jax import lax
from jax.experimental import pallas as pl
from jax.experimental.pallas import tpu as pltpu
from jax.experimental.pallas import tpu_sc as plsc

N_NODES = 10000
N_EDGES = 320000
F_IN = 128
F_HID = 128
N_CLASSES = 47
C_PAD = 128

NC = 2
NS = 16
NW = NC * NS
EDGES_PER_W = N_EDGES // NW
CHUNK = 80
N_ITERS = EDGES_PER_W // CHUNK
N_PAD = 10240
ROWS_PER_TILE = N_PAD // NS

_sc_mesh = plsc.VectorSubcoreMesh(core_axis_name="c", subcore_axis_name="s",
                                  num_cores=NC, num_subcores=NS)

W = F_HID


def _sc_agg_body(g_hbm, src_hbm, dst_hbm, zrows_hbm, out_hbm, sidx_v, didx_v,
                 rows_v, acc_sh, sem):
    cid = lax.axis_index("c")
    tid = lax.axis_index("s")
    wid = cid * NS + tid

    pltpu.sync_copy(zrows_hbm, acc_sh.at[pl.ds(tid * ROWS_PER_TILE, ROWS_PER_TILE)])
    plsc.subcore_barrier()

    base = wid * EDGES_PER_W

    def body(i, carry):
        off = base + i * CHUNK
        pltpu.sync_copy(src_hbm.at[pl.ds(off, CHUNK)], sidx_v)
        pltpu.sync_copy(dst_hbm.at[pl.ds(off, CHUNK)], didx_v)
        pltpu.async_copy(g_hbm.at[sidx_v], rows_v, sem).wait()
        pltpu.sync_copy(rows_v, acc_sh.at[didx_v], add=True)
        return carry

    lax.fori_loop(0, N_ITERS, body, 0)
    plsc.subcore_barrier()

    sl = pl.ds(tid * ROWS_PER_TILE, ROWS_PER_TILE)
    pltpu.sync_copy(acc_sh.at[sl], out_hbm.at[cid, sl])


_sc_agg = pl.kernel(
    _sc_agg_body,
    out_type=[jax.ShapeDtypeStruct((NC, N_PAD, W), jnp.float32)],
    mesh=_sc_mesh,
    scratch_types=[
        pltpu.VMEM((CHUNK,), jnp.int32),
        pltpu.VMEM((CHUNK,), jnp.int32),
        pltpu.VMEM((CHUNK, W), jnp.float32),
        pltpu.VMEM_SHARED((N_PAD, W), jnp.float32),
        pltpu.SemaphoreType.DMA,
    ],
)


def _sc_deg_body(dst_hbm, zrows_hbm, ones_hbm, out_hbm, didx_v, ones_v,
                 acc_sh, sem):
    cid = lax.axis_index("c")
    tid = lax.axis_index("s")
    wid = cid * NS + tid

    pltpu.sync_copy(zrows_hbm, acc_sh.at[pl.ds(tid * ROWS_PER_TILE, ROWS_PER_TILE)])
    pltpu.sync_copy(ones_hbm, ones_v)
    plsc.subcore_barrier()

    base = wid * EDGES_PER_W

    def body(i, carry):
        off = base + i * CHUNK
        pltpu.sync_copy(dst_hbm.at[pl.ds(off, CHUNK)], didx_v)
        pltpu.sync_copy(ones_v, acc_sh.at[didx_v], add=True)
        return carry

    lax.fori_loop(0, N_ITERS, body, 0)
    plsc.subcore_barrier()

    sl = pl.ds(tid * ROWS_PER_TILE, ROWS_PER_TILE)
    pltpu.sync_copy(acc_sh.at[sl], out_hbm.at[cid, sl])


_sc_deg = pl.kernel(
    _sc_deg_body,
    out_type=[jax.ShapeDtypeStruct((NC, N_PAD, W), jnp.float32)],
    mesh=_sc_mesh,
    scratch_types=[
        pltpu.VMEM((CHUNK,), jnp.int32),
        pltpu.VMEM((CHUNK, W), jnp.float32),
        pltpu.VMEM_SHARED((N_PAD, W), jnp.float32),
        pltpu.SemaphoreType.DMA,
    ],
)

ROW_BLK = 1000
GRID = N_NODES // ROW_BLK


def _mm_first_body(x_ref, ws_ref, wn_ref, b_ref, s_ref, g_ref):
    x = x_ref[...]
    s_ref[...] = jnp.dot(x, ws_ref[...], preferred_element_type=jnp.float32) + b_ref[...]
    g_ref[...] = jnp.dot(x, wn_ref[...], preferred_element_type=jnp.float32)


def _mm_mid_body(s_ref, pa_ref, pb_ref, da_ref, db_ref, ws_ref, wn_ref,
                 b_ref, s_out_ref, g_out_ref):
    deg = da_ref[..., 0:1] + db_ref[..., 0:1]
    inv = 1.0 / jnp.maximum(deg, 1.0)
    h = jnp.maximum(s_ref[...] + (pa_ref[...] + pb_ref[...]) * inv, 0.0)
    s_out_ref[...] = jnp.dot(h, ws_ref[...], preferred_element_type=jnp.float32) + b_ref[...]
    g_out_ref[...] = jnp.dot(h, wn_ref[...], preferred_element_type=jnp.float32)


def _final_body(s_ref, pa_ref, pb_ref, da_ref, db_ref, out_ref):
    deg = da_ref[..., 0:1] + db_ref[..., 0:1]
    inv = 1.0 / jnp.maximum(deg, 1.0)
    res = s_ref[...] + (pa_ref[...] + pb_ref[...]) * inv
    out_ref[...] = res[:, :N_CLASSES]


def _row_spec(width):
    return pl.BlockSpec((ROW_BLK, width), lambda i: (i, 0))


def _full_spec(shape):
    ndim = len(shape)
    return pl.BlockSpec(shape, lambda i: (0,) * ndim)


def kernel(x, edge_index, Wself0, Wneigh0, b0, Wself1, Wneigh1, b1,
           Wself2, Wneigh2, b2):
    src = edge_index[0].astype(jnp.int32)
    dst = edge_index[1].astype(jnp.int32)

    zrows = jnp.zeros((ROWS_PER_TILE, W), jnp.float32)
    ones = jnp.ones((CHUNK, W), jnp.float32)

    wn2 = jnp.zeros((F_HID, C_PAD), jnp.float32).at[:, :N_CLASSES].set(Wneigh2)
    ws2 = jnp.zeros((F_HID, C_PAD), jnp.float32).at[:, :N_CLASSES].set(Wself2)
    b2p = jnp.zeros((C_PAD,), jnp.float32).at[:N_CLASSES].set(b2)

    degp = _sc_deg(dst, zrows, ones)[0]
    da, db = degp[0], degp[1]

    s0, g0 = pl.pallas_call(
        _mm_first_body,
        grid=(GRID,),
        in_specs=[_row_spec(F_IN), _full_spec((F_IN, F_HID)),
                  _full_spec((F_IN, F_HID)), _full_spec((1, F_HID))],
        out_specs=[_row_spec(F_HID), _row_spec(F_HID)],
        out_shape=[jax.ShapeDtypeStruct((N_NODES, F_HID), jnp.float32)] * 2,
    )(x, Wself0, Wneigh0, b0.reshape(1, F_HID))

    p0 = _sc_agg(g0, src, dst, zrows)[0]

    s1, g1 = pl.pallas_call(
        _mm_mid_body,
        grid=(GRID,),
        in_specs=[_row_spec(F_HID), _row_spec(F_HID), _row_spec(F_HID),
                  _row_spec(W), _row_spec(W),
                  _full_spec((F_HID, F_HID)), _full_spec((F_HID, F_HID)),
                  _full_spec((1, F_HID))],
        out_specs=[_row_spec(F_HID), _row_spec(F_HID)],
        out_shape=[jax.ShapeDtypeStruct((N_NODES, F_HID), jnp.float32)] * 2,
    )(s0, p0[0], p0[1], da, db, Wself1, Wneigh1, b1.reshape(1, F_HID))

    p1 = _sc_agg(g1, src, dst, zrows)[0]

    s2, g2 = pl.pallas_call(
        _mm_mid_body,
        grid=(GRID,),
        in_specs=[_row_spec(F_HID), _row_spec(F_HID), _row_spec(F_HID),
                  _row_spec(W), _row_spec(W),
                  _full_spec((F_HID, C_PAD)), _full_spec((F_HID, C_PAD)),
                  _full_spec((1, C_PAD))],
        out_specs=[_row_spec(C_PAD), _row_spec(C_PAD)],
        out_shape=[jax.ShapeDtypeStruct((N_NODES, C_PAD), jnp.float32)] * 2,
    )(s1, p1[0], p1[1], da, db, ws2, wn2, b2p.reshape(1, C_PAD))

    p2 = _sc_agg(g2, src, dst, zrows)[0]

    out = pl.pallas_call(
        _final_body,
        grid=(GRID,),
        in_specs=[_row_spec(C_PAD), _row_spec(C_PAD), _row_spec(C_PAD),
                  _row_spec(W), _row_spec(W)],
        out_specs=_row_spec(N_CLASSES),
        out_shape=jax.ShapeDtypeStruct((N_NODES, N_CLASSES), jnp.float32),
    )(s2, p2[0], p2[1], da, db)

    return out
